# Initial kernel scaffold; baseline (speedup 1.0000x reference)
import functools

import jax
import jax.numpy as jnp
from jax.experimental import pallas as pl
from jax.experimental.pallas import tpu as pltpu


def _round_up(a, b):
    return (a + b - 1) // b * b


def _conv_windows(xw_ref, w_bf, wout, stride, ksz):
    cin = xw_ref.shape[1]
    tile_r = xw_ref.shape[2]
    for wo in range(wout):
        xwin = xw_ref[pl.ds(wo * stride, ksz)].reshape(ksz * cin, tile_r)
        yield wo, jnp.dot(w_bf, xwin, preferred_element_type=jnp.float32)


def _stats_kernel(xw_ref, w_ref, psum_ref, psq_ref, acc_s, acc_q,
                  *, stride, wout, ksz, n_tiles):
    t = pl.program_id(1)
    cout = w_ref.shape[0]
    tile_r = xw_ref.shape[2]

    @pl.when(t == 0)
    def _init():
        acc_s[...] = jnp.zeros_like(acc_s)
        acc_q[...] = jnp.zeros_like(acc_q)

    w_bf = w_ref[...].astype(jnp.bfloat16)
    ty = jnp.zeros((cout, tile_r), jnp.float32)
    tq = jnp.zeros((cout, tile_r), jnp.float32)
    for _, y in _conv_windows(xw_ref, w_bf, wout, stride, ksz):
        ty = ty + y
        tq = tq + y * y
    acc_s[...] += jnp.sum(ty, axis=1, keepdims=True)
    acc_q[...] += jnp.sum(tq, axis=1, keepdims=True)

    @pl.when(t == n_tiles - 1)
    def _flush():
        psum_ref[0] = acc_s[...]
        psq_ref[0] = acc_q[...]


def _norm_kernel(xw_ref, w_ref, g_ref, b_ref, psum_ref, psq_ref, o_ref,
                 *, stride, wout, ksz, inv_m, eps):
    cout = o_ref.shape[1]
    tile_r = o_ref.shape[2]

    s = jnp.sum(psum_ref[...], axis=0)
    q = jnp.sum(psq_ref[...], axis=0)
    mean = s * inv_m
    var = q * inv_m - mean * mean
    scale = g_ref[...] * jax.lax.rsqrt(var + eps)
    shift = b_ref[...] - mean * scale
    w_bf = (w_ref[...] * scale).astype(jnp.bfloat16)
    shift_b = jnp.broadcast_to(shift, (cout, tile_r))
    for wo, y in _conv_windows(xw_ref, w_bf, wout, stride, ksz):
        o_ref[wo] = jnp.maximum(y + shift_b, 0.0)


def kernel(x, conv_w, conv_b, bn_gamma, bn_beta):
    del conv_b
    N, Cin, H, W = x.shape
    Cout = conv_w.shape[0]
    K = conv_w.shape[3]
    S = 2
    Wout = (W - K) // S + 1
    R = N * H
    eps = 1e-5

    NC = 2
    TILE_A = 512
    TILE_B = 256
    R_pad = _round_up(R, NC * max(TILE_A, TILE_B))
    nt_a = R_pad // (NC * TILE_A)
    nt_b = R_pad // (NC * TILE_B)

    xw = jnp.transpose(x, (3, 1, 0, 2)).reshape(W, Cin, R)
    xw = jnp.pad(xw, ((0, 0), (0, 0), (0, R_pad - R))).astype(jnp.bfloat16)

    w_mat = conv_w.reshape(Cout, Cin, K).transpose(0, 2, 1).reshape(Cout, K * Cin)
    w_mat = w_mat.astype(jnp.float32)

    g_col = bn_gamma.reshape(Cout, 1).astype(jnp.float32)
    b_col = bn_beta.reshape(Cout, 1).astype(jnp.float32)
    inv_m = 1.0 / float(N * H * Wout)

    stats = pl.pallas_call(
        functools.partial(_stats_kernel, stride=S, wout=Wout, ksz=K,
                          n_tiles=nt_a),
        out_shape=[jax.ShapeDtypeStruct((NC, Cout, 1), jnp.float32)] * 2,
        grid=(NC, nt_a),
        in_specs=[
            pl.BlockSpec((W, Cin, TILE_A), lambda c, i: (0, 0, c * nt_a + i)),
            pl.BlockSpec((Cout, K * Cin), lambda c, i: (0, 0)),
        ],
        out_specs=[pl.BlockSpec((1, Cout, 1), lambda c, i: (c, 0, 0))] * 2,
        scratch_shapes=[pltpu.VMEM((Cout, 1), jnp.float32),
                        pltpu.VMEM((Cout, 1), jnp.float32)],
        compiler_params=pltpu.CompilerParams(
            dimension_semantics=("core_parallel", "arbitrary"),
            vmem_limit_bytes=48 * 1024 * 1024,
        ),
        cost_estimate=pl.CostEstimate(
            flops=2 * R_pad * Wout * Cout * Cin * K,
            transcendentals=0,
            bytes_accessed=int(xw.size * 2 + w_mat.size * 4),
        ),
    )(xw, w_mat)
    psum, psq = stats

    out_wcr = pl.pallas_call(
        functools.partial(_norm_kernel, stride=S, wout=Wout, ksz=K,
                          inv_m=inv_m, eps=eps),
        out_shape=jax.ShapeDtypeStruct((Wout, Cout, R_pad), jnp.float32),
        grid=(NC, nt_b),
        in_specs=[
            pl.BlockSpec((W, Cin, TILE_B), lambda c, i: (0, 0, c * nt_b + i)),
            pl.BlockSpec((Cout, K * Cin), lambda c, i: (0, 0)),
            pl.BlockSpec((Cout, 1), lambda c, i: (0, 0)),
            pl.BlockSpec((Cout, 1), lambda c, i: (0, 0)),
            pl.BlockSpec((NC, Cout, 1), lambda c, i: (0, 0, 0)),
            pl.BlockSpec((NC, Cout, 1), lambda c, i: (0, 0, 0)),
        ],
        out_specs=pl.BlockSpec((Wout, Cout, TILE_B),
                               lambda c, i: (0, 0, c * nt_b + i)),
        compiler_params=pltpu.CompilerParams(
            dimension_semantics=("core_parallel", "arbitrary"),
            vmem_limit_bytes=48 * 1024 * 1024,
        ),
        cost_estimate=pl.CostEstimate(
            flops=2 * R_pad * Wout * Cout * Cin * K,
            transcendentals=Cout * NC * nt_b,
            bytes_accessed=int(xw.size * 2 + w_mat.size * 4
                               + Wout * Cout * R_pad * 4),
        ),
    )(xw, w_mat, g_col, b_col, psum, psq)

    out = out_wcr[:, :, :R]
    out = jnp.transpose(out, (2, 1, 0)).reshape(N, H, Cout, Wout)
    return jnp.transpose(out, (0, 2, 1, 3))


# speedup vs baseline: 8.4633x; 8.4633x over previous
import functools

import jax
import jax.numpy as jnp
from jax.experimental import pallas as pl
from jax.experimental.pallas import tpu as pltpu


def _round_up(a, b):
    return (a + b - 1) // b * b


def _conv_windows(xw_ref, w_bf, wout, stride, ksz):
    cin = xw_ref.shape[1]
    tile_r = xw_ref.shape[2]
    for wo in range(wout):
        xwin = xw_ref[pl.ds(wo * stride, ksz)].reshape(ksz * cin, tile_r)
        yield wo, jnp.dot(w_bf, xwin, preferred_element_type=jnp.float32)


def _fused_kernel(xw_ref, w_ref, g_ref, b_ref, o_ref, acc_s, acc_q,
                  *, stride, wout, ksz, inv_m, eps):
    sweep = pl.program_id(0)
    t = pl.program_id(1)
    cout = w_ref.shape[0]
    tile_r = xw_ref.shape[2]

    @pl.when(jnp.logical_and(sweep == 0, t == 0))
    def _init():
        acc_s[...] = jnp.zeros_like(acc_s)
        acc_q[...] = jnp.zeros_like(acc_q)

    @pl.when(sweep == 0)
    def _stats():
        w_bf = w_ref[...].astype(jnp.bfloat16)
        ty = jnp.zeros((cout, tile_r), jnp.float32)
        tq = jnp.zeros((cout, tile_r), jnp.float32)
        for _, y in _conv_windows(xw_ref, w_bf, wout, stride, ksz):
            ty = ty + y
            tq = tq + y * y
        acc_s[...] += jnp.sum(ty, axis=1, keepdims=True)
        acc_q[...] += jnp.sum(tq, axis=1, keepdims=True)

    @pl.when(sweep == 1)
    def _normalize():
        mean = acc_s[...] * inv_m
        var = acc_q[...] * inv_m - mean * mean
        scale = g_ref[...] * jax.lax.rsqrt(var + eps)
        shift = b_ref[...] - mean * scale
        w_bf = (w_ref[...] * scale).astype(jnp.bfloat16)
        shift_b = jnp.broadcast_to(shift, (cout, tile_r))
        for wo, y in _conv_windows(xw_ref, w_bf, wout, stride, ksz):
            o_ref[wo] = jnp.maximum(y + shift_b, 0.0)


def kernel(x, conv_w, conv_b, bn_gamma, bn_beta):
    del conv_b
    N, Cin, H, W = x.shape
    Cout = conv_w.shape[0]
    K = conv_w.shape[3]
    S = 2
    Wout = (W - K) // S + 1
    R = N * H
    eps = 1e-5

    TILE_R = 256
    R_pad = _round_up(R, TILE_R)
    n_tiles = R_pad // TILE_R

    xw = jnp.transpose(x, (3, 1, 0, 2)).reshape(W, Cin, R)
    xw = jnp.pad(xw, ((0, 0), (0, 0), (0, R_pad - R))).astype(jnp.bfloat16)

    w_mat = conv_w.reshape(Cout, Cin, K).transpose(0, 2, 1).reshape(Cout, K * Cin)
    w_mat = w_mat.astype(jnp.float32)

    g_col = bn_gamma.reshape(Cout, 1).astype(jnp.float32)
    b_col = bn_beta.reshape(Cout, 1).astype(jnp.float32)
    inv_m = 1.0 / float(N * H * Wout)

    out_wcr = pl.pallas_call(
        functools.partial(_fused_kernel, stride=S, wout=Wout, ksz=K,
                          inv_m=inv_m, eps=eps),
        out_shape=jax.ShapeDtypeStruct((Wout, Cout, R_pad), jnp.float32),
        grid=(2, n_tiles),
        in_specs=[
            pl.BlockSpec((W, Cin, TILE_R), lambda s, i: (0, 0, i)),
            pl.BlockSpec((Cout, K * Cin), lambda s, i: (0, 0)),
            pl.BlockSpec((Cout, 1), lambda s, i: (0, 0)),
            pl.BlockSpec((Cout, 1), lambda s, i: (0, 0)),
        ],
        out_specs=pl.BlockSpec((Wout, Cout, TILE_R),
                               lambda s, i: (0, 0, s * i)),
        scratch_shapes=[pltpu.VMEM((Cout, 1), jnp.float32),
                        pltpu.VMEM((Cout, 1), jnp.float32)],
        compiler_params=pltpu.CompilerParams(
            dimension_semantics=("arbitrary", "arbitrary"),
            vmem_limit_bytes=48 * 1024 * 1024,
        ),
        cost_estimate=pl.CostEstimate(
            flops=2 * 2 * R_pad * Wout * Cout * Cin * K,
            transcendentals=Cout * n_tiles,
            bytes_accessed=int(2 * xw.size * 2 + w_mat.size * 4
                               + Wout * Cout * R_pad * 4),
        ),
    )(xw, w_mat, g_col, b_col)

    out = out_wcr[:, :, :R]
    out = jnp.transpose(out, (2, 1, 0)).reshape(N, H, Cout, Wout)
    return jnp.transpose(out, (0, 2, 1, 3))
